# baseline (speedup 1.0000x reference)
import functools

import jax
import jax.numpy as jnp
from jax import lax
from jax.experimental import pallas as pl
from jax.experimental.pallas import tpu as pltpu
from jax.experimental.pallas import tpu_sc as plsc

HEIGHT, WIDTH, DIM, BATCH = 32, 32, 768, 64
LANES = 16
NC, NS = 2, 16
PART = WIDTH // 4

_mesh = plsc.VectorSubcoreMesh(core_axis_name="c", subcore_axis_name="s")


@functools.partial(
    pl.kernel,
    mesh=_mesh,
    out_type=jax.ShapeDtypeStruct((BATCH, HEIGHT * WIDTH, DIM), jnp.float32),
    scratch_types=[
        pltpu.VMEM((WIDTH, DIM), jnp.float32),
        pltpu.VMEM((DIM,), jnp.float32),
        pltpu.SemaphoreType.DMA,
    ],
)
def _sc_pos_kernel(row_hbm, col_hbm, out_hbm, buf_v, row_v, sem):
    wid = lax.axis_index("s") * NC + lax.axis_index("c")
    ccol = pltpu.async_copy(col_hbm, buf_v, sem)
    crow = pltpu.async_copy(row_hbm.at[wid], row_v, sem)
    ccol.wait()
    crow.wait()

    def add_row(w, carry):
        for j in range(DIM // LANES):
            sl = pl.ds(j * LANES, LANES)
            buf_v[w, sl] = buf_v[w, sl] + row_v[sl]
        return carry

    base = wid * WIDTH
    group = 16
    pending = []

    def stream_part(lo):
        for g in range(BATCH // group):
            cur = [
                pltpu.async_copy(
                    buf_v.at[pl.ds(lo, PART)],
                    out_hbm.at[b, pl.ds(base + lo, PART)],
                    sem,
                )
                for b in range(g * group, (g + 1) * group)
            ]
            if pending:
                for c in pending.pop():
                    c.wait()
            pending.append(cur)

    for q in range(WIDTH // PART):
        lax.fori_loop(q * PART, (q + 1) * PART, add_row, 0)
        stream_part(q * PART)
    while pending:
        for c in pending.pop():
            c.wait()


def kernel(batch_size, row_embed, col_embed):
    del batch_size
    return _sc_pos_kernel(row_embed, col_embed)


# speedup vs baseline: 1.0185x; 1.0185x over previous
import functools

import jax
import jax.numpy as jnp
from jax import lax
from jax.experimental import pallas as pl
from jax.experimental.pallas import tpu as pltpu
from jax.experimental.pallas import tpu_sc as plsc

HEIGHT, WIDTH, DIM, BATCH = 32, 32, 768, 64
LANES = 16
NC, NS = 2, 16
PART = WIDTH // 2

_mesh = plsc.VectorSubcoreMesh(core_axis_name="c", subcore_axis_name="s")


@functools.partial(
    pl.kernel,
    mesh=_mesh,
    out_type=jax.ShapeDtypeStruct((BATCH, HEIGHT * WIDTH, DIM), jnp.float32),
    scratch_types=[
        pltpu.VMEM((WIDTH, DIM), jnp.float32),
        pltpu.VMEM((DIM,), jnp.float32),
        pltpu.SemaphoreType.DMA,
    ],
)
def _sc_pos_kernel(row_hbm, col_hbm, out_hbm, buf_v, row_v, sem):
    wid = lax.axis_index("s") * NC + lax.axis_index("c")
    ccol = pltpu.async_copy(col_hbm, buf_v, sem)
    crow = pltpu.async_copy(row_hbm.at[wid], row_v, sem)
    ccol.wait()
    crow.wait()

    def add_row(w, carry):
        for j in range(DIM // LANES):
            sl = pl.ds(j * LANES, LANES)
            buf_v[w, sl] = buf_v[w, sl] + row_v[sl]
        return carry

    base = wid * WIDTH
    group = 16
    pending = []

    def stream_part(lo):
        for g in range(BATCH // group):
            cur = [
                pltpu.async_copy(
                    buf_v.at[pl.ds(lo, PART)],
                    out_hbm.at[b, pl.ds(base + lo, PART)],
                    sem,
                )
                for b in range(g * group, (g + 1) * group)
            ]
            if pending:
                for c in pending.pop():
                    c.wait()
            pending.append(cur)

    for q in range(WIDTH // PART):
        lax.fori_loop(q * PART, (q + 1) * PART, add_row, 0)
        stream_part(q * PART)
    while pending:
        for c in pending.pop():
            c.wait()


def kernel(batch_size, row_embed, col_embed):
    del batch_size
    return _sc_pos_kernel(row_embed, col_embed)


# speedup vs baseline: 1.0355x; 1.0168x over previous
import functools

import jax
import jax.numpy as jnp
from jax import lax
from jax.experimental import pallas as pl
from jax.experimental.pallas import tpu as pltpu
from jax.experimental.pallas import tpu_sc as plsc

HEIGHT, WIDTH, DIM, BATCH = 32, 32, 768, 64
LANES = 16
NC, NS = 2, 16
HALF = WIDTH // 2
TC_B = 32
SC_B = BATCH - TC_B

_mesh = plsc.VectorSubcoreMesh(core_axis_name="c", subcore_axis_name="s")


@functools.partial(
    pl.kernel,
    mesh=_mesh,
    out_type=jax.ShapeDtypeStruct((BATCH, HEIGHT * WIDTH, DIM), jnp.float32),
    scratch_types=[
        pltpu.VMEM((WIDTH, DIM), jnp.float32),
        pltpu.VMEM((DIM,), jnp.float32),
        pltpu.SemaphoreType.DMA,
    ],
)
def _sc_pos_kernel(row_hbm, col_hbm, out_hbm, buf_v, row_v, sem):
    wid = lax.axis_index("s") * NC + lax.axis_index("c")
    ccol = pltpu.async_copy(col_hbm, buf_v, sem)
    crow = pltpu.async_copy(row_hbm.at[wid], row_v, sem)
    ccol.wait()
    crow.wait()

    def add_row(w, carry):
        for j in range(DIM // LANES):
            sl = pl.ds(j * LANES, LANES)
            buf_v[w, sl] = buf_v[w, sl] + row_v[sl]
        return carry

    base = wid * WIDTH
    group = 16
    pending = []

    def stream_half(lo):
        for g in range((SC_B + group - 1) // group):
            cur = [
                pltpu.async_copy(
                    buf_v.at[pl.ds(lo, HALF)],
                    out_hbm.at[b, pl.ds(base + lo, HALF)],
                    sem,
                )
                for b in range(
                    TC_B + g * group, min(TC_B + (g + 1) * group, BATCH)
                )
            ]
            if pending:
                for c in pending.pop():
                    c.wait()
            pending.append(cur)

    lax.fori_loop(0, HALF, add_row, 0)
    stream_half(0)
    lax.fori_loop(HALF, WIDTH, add_row, 0)
    stream_half(HALF)
    while pending:
        for c in pending.pop():
            c.wait()


def _tc_pos_kernel(alias_ref, row_ref, col_ref, out_ref):
    del alias_ref
    r = row_ref[:]
    c = col_ref[:]
    pos = (r[:, None, :] + c[None, :, :]).reshape(HEIGHT * WIDTH, DIM)
    out_ref[0] = pos


def kernel(batch_size, row_embed, col_embed):
    del batch_size
    sc_out = _sc_pos_kernel(row_embed, col_embed)
    return pl.pallas_call(
        _tc_pos_kernel,
        grid=(TC_B,),
        in_specs=[
            pl.BlockSpec(memory_space=pltpu.MemorySpace.HBM),
            pl.BlockSpec((HEIGHT, DIM), lambda b: (0, 0)),
            pl.BlockSpec((WIDTH, DIM), lambda b: (0, 0)),
        ],
        out_specs=pl.BlockSpec((1, HEIGHT * WIDTH, DIM), lambda b: (b, 0, 0)),
        out_shape=jax.ShapeDtypeStruct((BATCH, HEIGHT * WIDTH, DIM), jnp.float32),
        input_output_aliases={0: 0},
    )(sc_out, row_embed, col_embed)
